# Initial kernel scaffold; baseline (speedup 1.0000x reference)
import jax
import jax.numpy as jnp
from jax.experimental import pallas as pl


def kernel(x, edge_index, batch, W1, as1, ad1, b1, W2, as2, ad2, b2, W3, as3, ad3, b3, W4, as4, ad4, b4, fcW, fcb):
    raise NotImplementedError("write your pallas kernel here")


# speedup vs baseline: 23.9986x; 23.9986x over previous
import functools

import jax
import jax.numpy as jnp
from jax import lax
from jax.experimental import pallas as pl
from jax.experimental.pallas import tpu as pltpu
from jax.experimental.pallas import tpu_sc as plsc

N = 10000
NP = 10240
E = 320000
ROWS = 1280
NEG = -1.0e30


def _k1_first(x_hbm, W3, as3, ad3):
    F = x_hbm.shape[1]
    C2 = W3.shape[2]
    Cout = C2 // 2

    def body(xr, w, asr, adr, hp_ref, as_ref, ad_ref, mask_ref):
        xb = xr[...]
        msum = jnp.sum(jnp.abs(xb), axis=1)
        valid = (msum > 1e-5).astype(jnp.float32)
        mask_ref[...] = valid[None, None, :]
        hp = jnp.dot(xb, w[0], preferred_element_type=jnp.float32)
        hp_ref[...] = hp[None]
        att_s = asr[0]
        att_d = adr[0]
        a_s = jnp.stack([
            jnp.sum(hp[:, :Cout] * att_s[0:1, :], axis=1),
            jnp.sum(hp[:, Cout:] * att_s[1:2, :], axis=1)], axis=0)
        a_d = jnp.stack([
            jnp.sum(hp[:, :Cout] * att_d[0:1, :], axis=1),
            jnp.sum(hp[:, Cout:] * att_d[1:2, :], axis=1)], axis=0)
        vb = valid[None, :] > 0.5
        as_ref[...] = jnp.where(vb, a_s, NEG)[None]
        ad_ref[...] = jnp.where(vb, a_d, NEG)[None]

    return pl.pallas_call(
        body,
        grid=(2, 8),
        in_specs=[
            pl.BlockSpec((ROWS, F), lambda h, i: (i, 0)),
            pl.BlockSpec((1, F, C2), lambda h, i: (h, 0, 0)),
            pl.BlockSpec((1, 2, Cout), lambda h, i: (h, 0, 0)),
            pl.BlockSpec((1, 2, Cout), lambda h, i: (h, 0, 0)),
        ],
        out_specs=[
            pl.BlockSpec((1, ROWS, C2), lambda h, i: (h, i, 0)),
            pl.BlockSpec((1, 2, ROWS), lambda h, i: (h, 0, i)),
            pl.BlockSpec((1, 2, ROWS), lambda h, i: (h, 0, i)),
            pl.BlockSpec((1, 1, ROWS), lambda h, i: (i, 0, 0)),
        ],
        out_shape=[
            jax.ShapeDtypeStruct((2, NP, C2), jnp.float32),
            jax.ShapeDtypeStruct((2, 2, NP), jnp.float32),
            jax.ShapeDtypeStruct((2, 2, NP), jnp.float32),
            jax.ShapeDtypeStruct((8, 1, ROWS), jnp.float32),
        ],
    )(x_hbm, W3, as3, ad3)


def _k1_mid(acc2, mask, b_prev, W3, as3, ad3):
    Cp2 = acc2.shape[1]
    C2 = W3.shape[2]
    Cout = C2 // 2

    def body(in0, in1, mref, bref, w, asr, adr, hp_ref, as_ref, ad_ref):
        z0 = in0[...] + bref[:, :Cp2]
        p0 = jnp.where(z0 >= 0, z0, 0.2 * z0)
        z1 = in1[...] + bref[:, Cp2:]
        p1 = jnp.where(z1 >= 0, z1, 0.2 * z1)
        wb = w[0]
        hp = jnp.dot(p0, wb[:Cp2, :], preferred_element_type=jnp.float32)
        hp = hp + jnp.dot(p1, wb[Cp2:, :], preferred_element_type=jnp.float32)
        hp_ref[...] = hp[None]
        att_s = asr[0]
        att_d = adr[0]
        a_s = jnp.stack([
            jnp.sum(hp[:, :Cout] * att_s[0:1, :], axis=1),
            jnp.sum(hp[:, Cout:] * att_s[1:2, :], axis=1)], axis=0)
        a_d = jnp.stack([
            jnp.sum(hp[:, :Cout] * att_d[0:1, :], axis=1),
            jnp.sum(hp[:, Cout:] * att_d[1:2, :], axis=1)], axis=0)
        vb = mref[0] > 0.5
        as_ref[...] = jnp.where(vb, a_s, NEG)[None]
        ad_ref[...] = jnp.where(vb, a_d, NEG)[None]

    Cp = 2 * Cp2
    return pl.pallas_call(
        body,
        grid=(2, 8),
        in_specs=[
            pl.BlockSpec((ROWS, Cp2), lambda h, i: (i, 0)),
            pl.BlockSpec((ROWS, Cp2), lambda h, i: (8 + i, 0)),
            pl.BlockSpec((1, 1, ROWS), lambda h, i: (i, 0, 0)),
            pl.BlockSpec((1, Cp), lambda h, i: (0, 0)),
            pl.BlockSpec((1, Cp, C2), lambda h, i: (h, 0, 0)),
            pl.BlockSpec((1, 2, Cout), lambda h, i: (h, 0, 0)),
            pl.BlockSpec((1, 2, Cout), lambda h, i: (h, 0, 0)),
        ],
        out_specs=[
            pl.BlockSpec((1, ROWS, C2), lambda h, i: (h, i, 0)),
            pl.BlockSpec((1, 2, ROWS), lambda h, i: (h, 0, i)),
            pl.BlockSpec((1, 2, ROWS), lambda h, i: (h, 0, i)),
        ],
        out_shape=[
            jax.ShapeDtypeStruct((2, NP, C2), jnp.float32),
            jax.ShapeDtypeStruct((2, 2, NP), jnp.float32),
            jax.ShapeDtypeStruct((2, 2, NP), jnp.float32),
        ],
    )(acc2, acc2, mask, b_prev, W3, as3, ad3)


_MESH = plsc.VectorSubcoreMesh(core_axis_name="c", subcore_axis_name="s")
_B2 = 400
_EB2 = E // 32


def _all_max(ref):
    def body(j, m):
        return jnp.maximum(m, ref[pl.ds(j * 16, 16)])
    m = lax.fori_loop(0, NP // 16, body,
                      jnp.full((16,), -3.0e38, jnp.float32))
    lane = lax.iota(jnp.int32, 16)
    dn = lax.GatherDimensionNumbers(offset_dims=(), collapsed_slice_dims=(0,),
                                    start_index_map=(0,))
    for kk in (8, 4, 2, 1):
        idx = lax.rem(lane + kk, 16)
        rot = lax.gather(m, idx[:, None], dn, (1,),
                         mode=lax.GatherScatterMode.PROMISE_IN_BOUNDS,
                         unique_indices=True)
        m = jnp.maximum(m, rot)
    return m


@functools.partial(
    pl.kernel,
    mesh=_MESH,
    compiler_params=pltpu.CompilerParams(needs_layout_passes=False),
    out_type=[
        jax.ShapeDtypeStruct((4 * E,), jnp.float32),
        jax.ShapeDtypeStruct((2 * 4 * NP,), jnp.float32),
    ],
    scratch_types=[
        pltpu.VMEM((NP,), jnp.float32),
        pltpu.VMEM((NP,), jnp.float32),
        pltpu.VMEM((NP,), jnp.float32),
        pltpu.VMEM((_B2,), jnp.int32),
        pltpu.VMEM((_B2,), jnp.int32),
        pltpu.VMEM((_B2,), jnp.float32),
        pltpu.VMEM((1280,), jnp.float32),
        pltpu.VMEM((1280,), jnp.float32),
        pltpu.VMEM_SHARED((16, NP), jnp.float32),
    ],
)
def _k2(asrc_hbm, adst_hbm, src_hbm, dst_hbm, exf_hbm, denp_hbm,
        asrc_vm, adst_vm, den_vm, srcv, dstv, exst, reda, redb, sp_den):
    c = lax.axis_index("c")
    s = lax.axis_index("s")
    h = lax.rem(s, 4)
    g = s // 4

    pltpu.sync_copy(asrc_hbm.at[pl.ds(h * NP, NP)], asrc_vm)
    pltpu.sync_copy(adst_hbm.at[pl.ds(h * NP, NP)], adst_vm)

    zv = jnp.zeros((16,), jnp.float32)

    def zbody(j, _):
        den_vm[pl.ds(j * 16, 16)] = zv
        return 0
    lax.fori_loop(0, NP // 16, zbody, 0)

    M = jnp.maximum(_all_max(asrc_vm) + _all_max(adst_vm), 0.0)

    t0 = c * (E // 2) + g * (E // 8)

    def chunk(k, _):
        base = t0 + k * _B2
        pltpu.sync_copy(src_hbm.at[pl.ds(base, _B2)], srcv)
        pltpu.sync_copy(dst_hbm.at[pl.ds(base, _B2)], dstv)

        def vec(j, _):
            s16 = srcv[pl.ds(j * 16, 16)]
            d16 = dstv[pl.ds(j * 16, 16)]
            ag = plsc.load_gather(asrc_vm, [s16])
            dg = plsc.load_gather(adst_vm, [d16])
            z = ag + dg
            e = jnp.where(z >= 0, z, 0.2 * z)
            ex = jnp.exp(e - M)
            exst[pl.ds(j * 16, 16)] = ex
            plsc.addupdate_scatter(den_vm, [d16], ex)
            return 0
        lax.fori_loop(0, _B2 // 16, vec, 0)
        pltpu.sync_copy(exst, exf_hbm.at[pl.ds(h * E + base, _B2)])
        return 0
    lax.fori_loop(0, (E // 8) // _B2, chunk, 0)

    pltpu.sync_copy(den_vm, sp_den.at[s])
    plsc.subcore_barrier()

    for r in range(2):
        bcol = g * 2560 + r * 1280
        pltpu.sync_copy(sp_den.at[h, pl.ds(bcol, 1280)], reda)
        for kpart in range(1, 4):
            pltpu.sync_copy(sp_den.at[h + 4 * kpart, pl.ds(bcol, 1280)],
                            redb)

            def addb(j, _):
                reda[pl.ds(j * 16, 16)] = (reda[pl.ds(j * 16, 16)]
                                           + redb[pl.ds(j * 16, 16)])
                return 0
            lax.fori_loop(0, 80, addb, 0)
        pltpu.sync_copy(
            reda, denp_hbm.at[pl.ds(c * 4 * NP + h * NP + bcol, 1280)])


_B4 = 80
_EB4 = E // 16
_RT = NP // 16


def _make_k4(C2, Cout):
    nq = C2 // 16

    @functools.partial(
        pl.kernel,
        mesh=_MESH,
        compiler_params=pltpu.CompilerParams(
            needs_layout_passes=False, use_tc_tiling_on_sc=False),
        out_type=jax.ShapeDtypeStruct((2 * NP, C2), jnp.float32),
        scratch_types=[
            pltpu.VMEM((2 * NP,), jnp.float32),
            pltpu.VMEM((_B4,), jnp.int32),
            pltpu.VMEM((_B4,), jnp.int32),
            pltpu.VMEM((_B4,), jnp.int32),
            pltpu.VMEM((_B4,), jnp.float32),
            pltpu.VMEM((_B4,), jnp.float32),
            pltpu.VMEM((_B4,), jnp.float32),
            pltpu.VMEM((_B4,), jnp.float32),
            pltpu.VMEM((_B4, C2), jnp.float32),
            pltpu.VMEM((_B4, C2), jnp.float32),
            pltpu.VMEM_SHARED((NP, C2), jnp.float32),
            pltpu.SemaphoreType.DMA,
        ],
    )
    def k4(hp_hbm, src_hbm, dst_hbm, exf_hbm, invd_hbm, acc_hbm,
           invd_vm, srcv, dstv, gidx, exc0, exc1, alp0, alp1,
           rows_v, out_v, acc_sp, sem):
        c = lax.axis_index("c")
        s = lax.axis_index("s")

        pltpu.sync_copy(invd_hbm.at[pl.ds(c * 2 * NP, 2 * NP)], invd_vm)

        zv = jnp.zeros((16,), jnp.float32)

        def zrow(r, _):
            for q in range(nq):
                out_v[r, pl.ds(q * 16, 16)] = zv
            return 0
        lax.fori_loop(0, _B4, zrow, 0)
        r0 = s * _RT

        def zchunk(k, _):
            pltpu.sync_copy(out_v, acc_sp.at[pl.ds(r0 + k * _B4, _B4)])
            return 0
        lax.fori_loop(0, _RT // _B4, zchunk, 0)
        plsc.subcore_barrier()

        t0 = s * _EB4
        coff = c * NP

        def chunk(k, _):
            base = t0 + k * _B4
            pltpu.sync_copy(src_hbm.at[pl.ds(base, _B4)], srcv)
            pltpu.sync_copy(dst_hbm.at[pl.ds(base, _B4)], dstv)
            exch = [exc0, exc1]
            alpv = [alp0, alp1]
            for h2 in range(2):
                pltpu.sync_copy(
                    exf_hbm.at[pl.ds((2 * c + h2) * E + base, _B4)],
                    exch[h2])

            def vec(j, _):
                s16 = srcv[pl.ds(j * 16, 16)]
                d16 = dstv[pl.ds(j * 16, 16)]
                gidx[pl.ds(j * 16, 16)] = s16 + coff
                for h2 in range(2):
                    ivd = plsc.load_gather(invd_vm, [d16 + h2 * NP])
                    alpv[h2][pl.ds(j * 16, 16)] = (
                        exch[h2][pl.ds(j * 16, 16)] * ivd)
                return 0
            lax.fori_loop(0, _B4 // 16, vec, 0)

            pltpu.async_copy(hp_hbm.at[gidx], rows_v, sem).wait()

            def scale(j, _):
                a0v = alpv[0][pl.ds(j * 16, 16)]
                a1v = alpv[1][pl.ds(j * 16, 16)]
                for e2 in range(16):
                    ed = j * 16 + e2
                    a0 = a0v[e2]
                    a1 = a1v[e2]
                    if Cout >= 16:
                        for q in range(nq):
                            av = a0 if q < Cout // 16 else a1
                            out_v[ed, pl.ds(q * 16, 16)] = (
                                rows_v[ed, pl.ds(q * 16, 16)] * av)
                    else:
                        lane = lax.iota(jnp.int32, 16)
                        av = jnp.where(lane < Cout,
                                       jnp.broadcast_to(a0, (16,)),
                                       jnp.broadcast_to(a1, (16,)))
                        out_v[ed, pl.ds(0, 16)] = (
                            rows_v[ed, pl.ds(0, 16)] * av)
                return 0
            lax.fori_loop(0, _B4 // 16, scale, 0)

            pltpu.sync_copy(out_v, acc_sp.at[dstv], add=True)
            return 0
        lax.fori_loop(0, _EB4 // _B4, chunk, 0)

        plsc.subcore_barrier()

        def wchunk(k, _):
            pltpu.sync_copy(acc_sp.at[pl.ds(r0 + k * _B4, _B4)],
                            acc_hbm.at[pl.ds(coff + r0 + k * _B4, _B4)])
            return 0
        lax.fori_loop(0, _RT // _B4, wchunk, 0)

    return k4


def _k5(acc2, mask, b4, batch2, fcW, fcb):
    def body(in0, in1, mref, bref, batr, wref, cref,
             out_ref, sums, counts):
        i = pl.program_id(0)
        z0 = in0[...] + bref[:, :16]
        p0 = jnp.where(z0 >= 0, z0, 0.2 * z0)
        z1 = in1[...] + bref[:, 16:]
        p1 = jnp.where(z1 >= 0, z1, 0.2 * z1)
        p = jnp.concatenate([p0, p1], axis=1)
        valid = mref[0] > 0.5
        bf = jnp.where(valid, batr[0], jnp.int32(16))
        gids = lax.broadcasted_iota(jnp.int32, (16, ROWS), 0)
        onehot = (gids == bf).astype(jnp.float32)
        contrib = jnp.dot(onehot, p, preferred_element_type=jnp.float32)
        cnt = jnp.sum(onehot, axis=1)

        @pl.when(i == 0)
        def _():
            sums[...] = jnp.zeros_like(sums)
            counts[...] = jnp.zeros_like(counts)

        sums[...] += contrib
        counts[...] += jnp.broadcast_to(cnt[:, None], counts.shape)

        @pl.when(i == 7)
        def _():
            pooled = sums[...] / jnp.maximum(counts[:, :1], 1.0)
            out_ref[...] = (
                jnp.dot(pooled, wref[...],
                        preferred_element_type=jnp.float32) + cref[...])

    return pl.pallas_call(
        body,
        grid=(8,),
        in_specs=[
            pl.BlockSpec((ROWS, 16), lambda i: (i, 0)),
            pl.BlockSpec((ROWS, 16), lambda i: (8 + i, 0)),
            pl.BlockSpec((1, 1, ROWS), lambda i: (i, 0, 0)),
            pl.BlockSpec((1, 32), lambda i: (0, 0)),
            pl.BlockSpec((1, 1, ROWS), lambda i: (i, 0, 0)),
            pl.BlockSpec((32, 16), lambda i: (0, 0)),
            pl.BlockSpec((1, 16), lambda i: (0, 0)),
        ],
        out_specs=pl.BlockSpec((16, 16), lambda i: (0, 0)),
        out_shape=jax.ShapeDtypeStruct((16, 16), jnp.float32),
        scratch_shapes=[
            pltpu.VMEM((16, 32), jnp.float32),
            pltpu.VMEM((16, 32), jnp.float32),
        ],
    )(acc2, acc2, mask, b4, batch2, fcW, fcb)


_K4 = {256: _make_k4(128, 64), 128: _make_k4(64, 32),
       64: _make_k4(32, 16), 32: _make_k4(16, 8)}


def _split_w(W):
    Cin, C = W.shape
    return W.reshape(Cin, 2, C // 2).transpose(1, 0, 2)


def _layer(hp, asrc3, adst3, src1, dst1):
    C2 = hp.shape[2]
    asrc = asrc3.reshape(4 * NP)
    adst = adst3.reshape(4 * NP)
    exf, denp = _k2(asrc, adst, src1, dst1)
    dpair = denp.reshape(2, 4 * NP)
    den = dpair[0] + dpair[1]
    invd = 1.0 / (den + 1e-16)
    hp2 = hp.reshape(2 * NP, C2)
    return _K4[2 * C2](hp2, src1, dst1, exf, invd)


def kernel(x, edge_index, batch, W1, as1, ad1, b1, W2, as2, ad2, b2,
           W3, as3, ad3, b3, W4, as4, ad4, b4, fcW, fcb):
    x_pad = jnp.zeros((NP, x.shape[1]), jnp.float32).at[:N].set(x)
    batch2 = (jnp.zeros((NP,), jnp.int32).at[:N].set(batch)
              .reshape(8, 1, ROWS))
    src1 = edge_index[0]
    dst1 = edge_index[1]

    hp, asrc3, adst3, mask = _k1_first(
        x_pad, _split_w(W1), as1.reshape(2, 2, -1), ad1.reshape(2, 2, -1))
    acc = _layer(hp, asrc3, adst3, src1, dst1)
    hp, asrc3, adst3 = _k1_mid(
        acc, mask, b1[None, :], _split_w(W2),
        as2.reshape(2, 2, -1), ad2.reshape(2, 2, -1))
    acc = _layer(hp, asrc3, adst3, src1, dst1)
    hp, asrc3, adst3 = _k1_mid(
        acc, mask, b2[None, :], _split_w(W3),
        as3.reshape(2, 2, -1), ad3.reshape(2, 2, -1))
    acc = _layer(hp, asrc3, adst3, src1, dst1)
    hp, asrc3, adst3 = _k1_mid(
        acc, mask, b3[None, :], _split_w(W4),
        as4.reshape(2, 2, -1), ad4.reshape(2, 2, -1))
    acc = _layer(hp, asrc3, adst3, src1, dst1)

    return _k5(acc, mask, b4[None, :], batch2, fcW, fcb[None, :])


# speedup vs baseline: 44.4537x; 1.8524x over previous
import functools

import jax
import jax.numpy as jnp
from jax import lax
from jax.experimental import pallas as pl
from jax.experimental.pallas import tpu as pltpu
from jax.experimental.pallas import tpu_sc as plsc

N = 10000
NP = 10240
E = 320000
ROWS = 1280
NEG = -1.0e30


def _k1_first(x_hbm, W3, as3, ad3):
    F = x_hbm.shape[1]
    C2 = W3.shape[2]
    Cout = C2 // 2

    def body(xr, w, asr, adr, hp_ref, as_ref, ad_ref, mask_ref):
        xb = xr[...]
        msum = jnp.sum(jnp.abs(xb), axis=1)
        valid = (msum > 1e-5).astype(jnp.float32)
        mask_ref[...] = valid[None, None, :]
        hp = jnp.dot(xb, w[0], preferred_element_type=jnp.float32)
        hp_ref[...] = hp[None]
        att_s = asr[0]
        att_d = adr[0]
        a_s = jnp.stack([
            jnp.sum(hp[:, :Cout] * att_s[0:1, :], axis=1),
            jnp.sum(hp[:, Cout:] * att_s[1:2, :], axis=1)], axis=0)
        a_d = jnp.stack([
            jnp.sum(hp[:, :Cout] * att_d[0:1, :], axis=1),
            jnp.sum(hp[:, Cout:] * att_d[1:2, :], axis=1)], axis=0)
        vb = valid[None, :] > 0.5
        as_ref[...] = jnp.where(vb, a_s, NEG)[None]
        ad_ref[...] = jnp.where(vb, a_d, NEG)[None]

    return pl.pallas_call(
        body,
        grid=(2, 8),
        in_specs=[
            pl.BlockSpec((ROWS, F), lambda h, i: (i, 0)),
            pl.BlockSpec((1, F, C2), lambda h, i: (h, 0, 0)),
            pl.BlockSpec((1, 2, Cout), lambda h, i: (h, 0, 0)),
            pl.BlockSpec((1, 2, Cout), lambda h, i: (h, 0, 0)),
        ],
        out_specs=[
            pl.BlockSpec((1, ROWS, C2), lambda h, i: (h, i, 0)),
            pl.BlockSpec((1, 2, ROWS), lambda h, i: (h, 0, i)),
            pl.BlockSpec((1, 2, ROWS), lambda h, i: (h, 0, i)),
            pl.BlockSpec((1, 1, ROWS), lambda h, i: (i, 0, 0)),
        ],
        out_shape=[
            jax.ShapeDtypeStruct((2, NP, C2), jnp.float32),
            jax.ShapeDtypeStruct((2, 2, NP), jnp.float32),
            jax.ShapeDtypeStruct((2, 2, NP), jnp.float32),
            jax.ShapeDtypeStruct((8, 1, ROWS), jnp.float32),
        ],
    )(x_hbm, W3, as3, ad3)


def _k1_mid(acc2, mask, b_prev, W3, as3, ad3):
    Cp2 = acc2.shape[1]
    C2 = W3.shape[2]
    Cout = C2 // 2

    def body(in0, in1, mref, bref, w, asr, adr, hp_ref, as_ref, ad_ref):
        z0 = in0[...] + bref[:, :Cp2]
        p0 = jnp.where(z0 >= 0, z0, 0.2 * z0)
        z1 = in1[...] + bref[:, Cp2:]
        p1 = jnp.where(z1 >= 0, z1, 0.2 * z1)
        wb = w[0]
        hp = jnp.dot(p0, wb[:Cp2, :], preferred_element_type=jnp.float32)
        hp = hp + jnp.dot(p1, wb[Cp2:, :], preferred_element_type=jnp.float32)
        hp_ref[...] = hp[None]
        att_s = asr[0]
        att_d = adr[0]
        a_s = jnp.stack([
            jnp.sum(hp[:, :Cout] * att_s[0:1, :], axis=1),
            jnp.sum(hp[:, Cout:] * att_s[1:2, :], axis=1)], axis=0)
        a_d = jnp.stack([
            jnp.sum(hp[:, :Cout] * att_d[0:1, :], axis=1),
            jnp.sum(hp[:, Cout:] * att_d[1:2, :], axis=1)], axis=0)
        vb = mref[0] > 0.5
        as_ref[...] = jnp.where(vb, a_s, NEG)[None]
        ad_ref[...] = jnp.where(vb, a_d, NEG)[None]

    Cp = 2 * Cp2
    return pl.pallas_call(
        body,
        grid=(2, 8),
        in_specs=[
            pl.BlockSpec((ROWS, Cp2), lambda h, i: (i, 0)),
            pl.BlockSpec((ROWS, Cp2), lambda h, i: (8 + i, 0)),
            pl.BlockSpec((1, 1, ROWS), lambda h, i: (i, 0, 0)),
            pl.BlockSpec((1, Cp), lambda h, i: (0, 0)),
            pl.BlockSpec((1, Cp, C2), lambda h, i: (h, 0, 0)),
            pl.BlockSpec((1, 2, Cout), lambda h, i: (h, 0, 0)),
            pl.BlockSpec((1, 2, Cout), lambda h, i: (h, 0, 0)),
        ],
        out_specs=[
            pl.BlockSpec((1, ROWS, C2), lambda h, i: (h, i, 0)),
            pl.BlockSpec((1, 2, ROWS), lambda h, i: (h, 0, i)),
            pl.BlockSpec((1, 2, ROWS), lambda h, i: (h, 0, i)),
        ],
        out_shape=[
            jax.ShapeDtypeStruct((2, NP, C2), jnp.float32),
            jax.ShapeDtypeStruct((2, 2, NP), jnp.float32),
            jax.ShapeDtypeStruct((2, 2, NP), jnp.float32),
        ],
    )(acc2, acc2, mask, b_prev, W3, as3, ad3)


_MESH = plsc.VectorSubcoreMesh(core_axis_name="c", subcore_axis_name="s")
_B2 = 400
_EB2 = E // 32


def _all_max(ref):
    def body(j, m):
        return jnp.maximum(m, ref[pl.ds(j * 16, 16)])
    m = lax.fori_loop(0, NP // 16, body,
                      jnp.full((16,), -3.0e38, jnp.float32))
    lane = lax.iota(jnp.int32, 16)
    dn = lax.GatherDimensionNumbers(offset_dims=(), collapsed_slice_dims=(0,),
                                    start_index_map=(0,))
    for kk in (8, 4, 2, 1):
        idx = lax.rem(lane + kk, 16)
        rot = lax.gather(m, idx[:, None], dn, (1,),
                         mode=lax.GatherScatterMode.PROMISE_IN_BOUNDS,
                         unique_indices=True)
        m = jnp.maximum(m, rot)
    return m


@functools.partial(
    pl.kernel,
    mesh=_MESH,
    compiler_params=pltpu.CompilerParams(needs_layout_passes=False),
    out_type=[
        jax.ShapeDtypeStruct((4 * E,), jnp.float32),
        jax.ShapeDtypeStruct((2 * 4 * NP,), jnp.float32),
    ],
    scratch_types=[
        pltpu.VMEM((NP,), jnp.float32),
        pltpu.VMEM((NP,), jnp.float32),
        pltpu.VMEM((NP,), jnp.float32),
        pltpu.VMEM((_B2,), jnp.int32),
        pltpu.VMEM((_B2,), jnp.int32),
        pltpu.VMEM((_B2,), jnp.float32),
        pltpu.VMEM((1280,), jnp.float32),
        pltpu.VMEM((1280,), jnp.float32),
        pltpu.VMEM_SHARED((16, NP), jnp.float32),
    ],
)
def _k2(asrc_hbm, adst_hbm, src_hbm, dst_hbm, exf_hbm, denp_hbm,
        asrc_vm, adst_vm, den_vm, srcv, dstv, exst, reda, redb, sp_den):
    c = lax.axis_index("c")
    s = lax.axis_index("s")
    h = lax.rem(s, 4)
    g = s // 4

    pltpu.sync_copy(asrc_hbm.at[pl.ds(h * NP, NP)], asrc_vm)
    pltpu.sync_copy(adst_hbm.at[pl.ds(h * NP, NP)], adst_vm)

    zv = jnp.zeros((16,), jnp.float32)

    def zbody(j, _):
        den_vm[pl.ds(j * 16, 16)] = zv
        return 0
    lax.fori_loop(0, NP // 16, zbody, 0)

    M = jnp.maximum(_all_max(asrc_vm) + _all_max(adst_vm), 0.0)

    t0 = c * (E // 2) + g * (E // 8)

    def chunk(k, _):
        base = t0 + k * _B2
        pltpu.sync_copy(src_hbm.at[pl.ds(base, _B2)], srcv)
        pltpu.sync_copy(dst_hbm.at[pl.ds(base, _B2)], dstv)

        def vec(j, _):
            s16 = srcv[pl.ds(j * 16, 16)]
            d16 = dstv[pl.ds(j * 16, 16)]
            ag = plsc.load_gather(asrc_vm, [s16])
            dg = plsc.load_gather(adst_vm, [d16])
            z = ag + dg
            e = jnp.where(z >= 0, z, 0.2 * z)
            ex = jnp.exp(e - M)
            exst[pl.ds(j * 16, 16)] = ex
            plsc.addupdate_scatter(den_vm, [d16], ex)
            return 0
        lax.fori_loop(0, _B2 // 16, vec, 0)
        pltpu.sync_copy(exst, exf_hbm.at[pl.ds(h * E + base, _B2)])
        return 0
    lax.fori_loop(0, (E // 8) // _B2, chunk, 0)

    pltpu.sync_copy(den_vm, sp_den.at[s])
    plsc.subcore_barrier()

    for r in range(2):
        bcol = g * 2560 + r * 1280
        pltpu.sync_copy(sp_den.at[h, pl.ds(bcol, 1280)], reda)
        for kpart in range(1, 4):
            pltpu.sync_copy(sp_den.at[h + 4 * kpart, pl.ds(bcol, 1280)],
                            redb)

            def addb(j, _):
                reda[pl.ds(j * 16, 16)] = (reda[pl.ds(j * 16, 16)]
                                           + redb[pl.ds(j * 16, 16)])
                return 0
            lax.fori_loop(0, 80, addb, 0)
        pltpu.sync_copy(
            reda, denp_hbm.at[pl.ds(c * 4 * NP + h * NP + bcol, 1280)])


_B4 = 80
_SUP = 2000
_EB4 = E // 16
_RT = NP // 16


def _make_k4(C2, Cout):
    nq = C2 // 16

    @functools.partial(
        pl.kernel,
        mesh=_MESH,
        compiler_params=pltpu.CompilerParams(
            needs_layout_passes=False, use_tc_tiling_on_sc=False),
        out_type=jax.ShapeDtypeStruct((2 * NP, C2), jnp.float32),
        scratch_types=[
            pltpu.VMEM((2 * NP,), jnp.float32),
            pltpu.VMEM((_SUP,), jnp.int32),
            pltpu.VMEM((_SUP,), jnp.int32),
            pltpu.VMEM((_SUP,), jnp.float32),
            pltpu.VMEM((_SUP,), jnp.float32),
            pltpu.VMEM((_B4,), jnp.int32),
            pltpu.VMEM((_B4,), jnp.int32),
            pltpu.VMEM((_B4,), jnp.float32),
            pltpu.VMEM((_B4,), jnp.float32),
            pltpu.VMEM((_B4, C2), jnp.float32),
            pltpu.VMEM_SHARED((NP, C2), jnp.float32),
            pltpu.SemaphoreType.DMA,
        ],
    )
    def k4(hp_hbm, src_hbm, dst_hbm, exf_hbm, invd_hbm, acc_hbm,
           invd_vm, srcB, dstB, exB0, exB1, gidx, dst_cur, alp0, alp1,
           rows_v, acc_sp, sem):
        c = lax.axis_index("c")
        s = lax.axis_index("s")

        pltpu.sync_copy(invd_hbm.at[pl.ds(c * 2 * NP, 2 * NP)], invd_vm)

        zv = jnp.zeros((16,), jnp.float32)

        def zrow(r, _):
            for q in range(nq):
                rows_v[r, pl.ds(q * 16, 16)] = zv
            return 0
        lax.fori_loop(0, _B4, zrow, 0)
        r0 = s * _RT

        def zchunk(k, _):
            pltpu.sync_copy(rows_v, acc_sp.at[pl.ds(r0 + k * _B4, _B4)])
            return 0
        lax.fori_loop(0, _RT // _B4, zchunk, 0)
        plsc.subcore_barrier()

        t0 = s * _EB4
        coff = c * NP
        exbs = [exB0, exB1]
        alps = [alp0, alp1]

        def sup(ks, _):
            base = t0 + ks * _SUP
            pltpu.sync_copy(src_hbm.at[pl.ds(base, _SUP)], srcB)
            pltpu.sync_copy(dst_hbm.at[pl.ds(base, _SUP)], dstB)
            for h2 in range(2):
                pltpu.sync_copy(
                    exf_hbm.at[pl.ds((2 * c + h2) * E + base, _SUP)],
                    exbs[h2])

            def sub(r, _):
                off = r * _B4

                def vec(j, _):
                    pos = off + j * 16
                    s16 = srcB[pl.ds(pos, 16)]
                    d16 = dstB[pl.ds(pos, 16)]
                    gidx[pl.ds(j * 16, 16)] = s16 + coff
                    dst_cur[pl.ds(j * 16, 16)] = d16
                    for h2 in range(2):
                        ivd = plsc.load_gather(invd_vm, [d16 + h2 * NP])
                        alps[h2][pl.ds(j * 16, 16)] = (
                            exbs[h2][pl.ds(pos, 16)] * ivd)
                    return 0
                lax.fori_loop(0, _B4 // 16, vec, 0)

                pltpu.async_copy(hp_hbm.at[gidx], rows_v, sem).wait()

                def scale(j, _):
                    a0v = alp0[pl.ds(j * 16, 16)]
                    a1v = alp1[pl.ds(j * 16, 16)]
                    for e2 in range(16):
                        ed = j * 16 + e2
                        a0 = a0v[e2]
                        a1 = a1v[e2]
                        if Cout >= 16:
                            for q in range(nq):
                                av = a0 if q < Cout // 16 else a1
                                rows_v[ed, pl.ds(q * 16, 16)] = (
                                    rows_v[ed, pl.ds(q * 16, 16)] * av)
                        else:
                            lane = lax.iota(jnp.int32, 16)
                            av = jnp.where(lane < Cout,
                                           jnp.broadcast_to(a0, (16,)),
                                           jnp.broadcast_to(a1, (16,)))
                            rows_v[ed, pl.ds(0, 16)] = (
                                rows_v[ed, pl.ds(0, 16)] * av)
                    return 0
                lax.fori_loop(0, _B4 // 16, scale, 0)

                pltpu.sync_copy(rows_v, acc_sp.at[dst_cur], add=True)
                return 0
            lax.fori_loop(0, _SUP // _B4, sub, 0)
            return 0
        lax.fori_loop(0, _EB4 // _SUP, sup, 0)

        plsc.subcore_barrier()

        def wchunk(k, _):
            pltpu.sync_copy(acc_sp.at[pl.ds(r0 + k * _B4, _B4)],
                            acc_hbm.at[pl.ds(coff + r0 + k * _B4, _B4)])
            return 0
        lax.fori_loop(0, _RT // _B4, wchunk, 0)

    return k4


def _k5(acc2, mask, b4, batch2, fcW, fcb):
    def body(in0, in1, mref, bref, batr, wref, cref,
             out_ref, sums, counts):
        i = pl.program_id(0)
        z0 = in0[...] + bref[:, :16]
        p0 = jnp.where(z0 >= 0, z0, 0.2 * z0)
        z1 = in1[...] + bref[:, 16:]
        p1 = jnp.where(z1 >= 0, z1, 0.2 * z1)
        p = jnp.concatenate([p0, p1], axis=1)
        valid = mref[0] > 0.5
        bf = jnp.where(valid, batr[0], jnp.int32(16))
        gids = lax.broadcasted_iota(jnp.int32, (16, ROWS), 0)
        onehot = (gids == bf).astype(jnp.float32)
        contrib = jnp.dot(onehot, p, preferred_element_type=jnp.float32)
        cnt = jnp.sum(onehot, axis=1)

        @pl.when(i == 0)
        def _():
            sums[...] = jnp.zeros_like(sums)
            counts[...] = jnp.zeros_like(counts)

        sums[...] += contrib
        counts[...] += jnp.broadcast_to(cnt[:, None], counts.shape)

        @pl.when(i == 7)
        def _():
            pooled = sums[...] / jnp.maximum(counts[:, :1], 1.0)
            out_ref[...] = (
                jnp.dot(pooled, wref[...],
                        preferred_element_type=jnp.float32) + cref[...])

    return pl.pallas_call(
        body,
        grid=(8,),
        in_specs=[
            pl.BlockSpec((ROWS, 16), lambda i: (i, 0)),
            pl.BlockSpec((ROWS, 16), lambda i: (8 + i, 0)),
            pl.BlockSpec((1, 1, ROWS), lambda i: (i, 0, 0)),
            pl.BlockSpec((1, 32), lambda i: (0, 0)),
            pl.BlockSpec((1, 1, ROWS), lambda i: (i, 0, 0)),
            pl.BlockSpec((32, 16), lambda i: (0, 0)),
            pl.BlockSpec((1, 16), lambda i: (0, 0)),
        ],
        out_specs=pl.BlockSpec((16, 16), lambda i: (0, 0)),
        out_shape=jax.ShapeDtypeStruct((16, 16), jnp.float32),
        scratch_shapes=[
            pltpu.VMEM((16, 32), jnp.float32),
            pltpu.VMEM((16, 32), jnp.float32),
        ],
    )(acc2, acc2, mask, b4, batch2, fcW, fcb)


_K4 = {256: _make_k4(128, 64), 128: _make_k4(64, 32),
       64: _make_k4(32, 16), 32: _make_k4(16, 8)}


def _split_w(W):
    Cin, C = W.shape
    return W.reshape(Cin, 2, C // 2).transpose(1, 0, 2)


def _layer(hp, asrc3, adst3, src1, dst1):
    C2 = hp.shape[2]
    asrc = asrc3.reshape(4 * NP)
    adst = adst3.reshape(4 * NP)
    exf, denp = _k2(asrc, adst, src1, dst1)
    dpair = denp.reshape(2, 4 * NP)
    den = dpair[0] + dpair[1]
    invd = 1.0 / (den + 1e-16)
    hp2 = hp.reshape(2 * NP, C2)
    return _K4[2 * C2](hp2, src1, dst1, exf, invd)


def kernel(x, edge_index, batch, W1, as1, ad1, b1, W2, as2, ad2, b2,
           W3, as3, ad3, b3, W4, as4, ad4, b4, fcW, fcb):
    x_pad = jnp.zeros((NP, x.shape[1]), jnp.float32).at[:N].set(x)
    batch2 = (jnp.zeros((NP,), jnp.int32).at[:N].set(batch)
              .reshape(8, 1, ROWS))
    src1 = edge_index[0]
    dst1 = edge_index[1]

    hp, asrc3, adst3, mask = _k1_first(
        x_pad, _split_w(W1), as1.reshape(2, 2, -1), ad1.reshape(2, 2, -1))
    acc = _layer(hp, asrc3, adst3, src1, dst1)
    hp, asrc3, adst3 = _k1_mid(
        acc, mask, b1[None, :], _split_w(W2),
        as2.reshape(2, 2, -1), ad2.reshape(2, 2, -1))
    acc = _layer(hp, asrc3, adst3, src1, dst1)
    hp, asrc3, adst3 = _k1_mid(
        acc, mask, b2[None, :], _split_w(W3),
        as3.reshape(2, 2, -1), ad3.reshape(2, 2, -1))
    acc = _layer(hp, asrc3, adst3, src1, dst1)
    hp, asrc3, adst3 = _k1_mid(
        acc, mask, b3[None, :], _split_w(W4),
        as4.reshape(2, 2, -1), ad4.reshape(2, 2, -1))
    acc = _layer(hp, asrc3, adst3, src1, dst1)

    return _k5(acc, mask, b4[None, :], batch2, fcW, fcb[None, :])


# speedup vs baseline: 51.0716x; 1.1489x over previous
import functools

import jax
import jax.numpy as jnp
from jax import lax
from jax.experimental import pallas as pl
from jax.experimental.pallas import tpu as pltpu
from jax.experimental.pallas import tpu_sc as plsc

N = 10000
NP = 10240
E = 320000
ROWS = 1280
NEG = -1.0e30


def _k1_first(x_hbm, W3, as3, ad3):
    F = x_hbm.shape[1]
    C2 = W3.shape[2]
    Cout = C2 // 2

    def body(xr, w, asr, adr, hp_ref, as_ref, ad_ref, mask_ref):
        xb = xr[...]
        msum = jnp.sum(jnp.abs(xb), axis=1)
        valid = (msum > 1e-5).astype(jnp.float32)
        mask_ref[...] = valid[None, None, :]
        hp = jnp.dot(xb, w[0], preferred_element_type=jnp.float32)
        hp_ref[...] = hp[None]
        att_s = asr[0]
        att_d = adr[0]
        a_s = jnp.stack([
            jnp.sum(hp[:, :Cout] * att_s[0:1, :], axis=1),
            jnp.sum(hp[:, Cout:] * att_s[1:2, :], axis=1)], axis=0)
        a_d = jnp.stack([
            jnp.sum(hp[:, :Cout] * att_d[0:1, :], axis=1),
            jnp.sum(hp[:, Cout:] * att_d[1:2, :], axis=1)], axis=0)
        vb = valid[None, :] > 0.5
        as_ref[...] = jnp.where(vb, a_s, NEG)[None]
        ad_ref[...] = jnp.where(vb, a_d, NEG)[None]

    return pl.pallas_call(
        body,
        grid=(2, 8),
        in_specs=[
            pl.BlockSpec((ROWS, F), lambda h, i: (i, 0)),
            pl.BlockSpec((1, F, C2), lambda h, i: (h, 0, 0)),
            pl.BlockSpec((1, 2, Cout), lambda h, i: (h, 0, 0)),
            pl.BlockSpec((1, 2, Cout), lambda h, i: (h, 0, 0)),
        ],
        out_specs=[
            pl.BlockSpec((1, ROWS, C2), lambda h, i: (h, i, 0)),
            pl.BlockSpec((1, 2, ROWS), lambda h, i: (h, 0, i)),
            pl.BlockSpec((1, 2, ROWS), lambda h, i: (h, 0, i)),
            pl.BlockSpec((1, 1, ROWS), lambda h, i: (i, 0, 0)),
        ],
        out_shape=[
            jax.ShapeDtypeStruct((2, NP, C2), jnp.float32),
            jax.ShapeDtypeStruct((2, 2, NP), jnp.float32),
            jax.ShapeDtypeStruct((2, 2, NP), jnp.float32),
            jax.ShapeDtypeStruct((8, 1, ROWS), jnp.float32),
        ],
    )(x_hbm, W3, as3, ad3)


def _k1_mid(acc2, mask, b_prev, W3, as3, ad3):
    Cp2 = acc2.shape[1]
    C2 = W3.shape[2]
    Cout = C2 // 2

    def body(in0, in1, mref, bref, w, asr, adr, hp_ref, as_ref, ad_ref):
        z0 = in0[...] + bref[:, :Cp2]
        p0 = jnp.where(z0 >= 0, z0, 0.2 * z0)
        z1 = in1[...] + bref[:, Cp2:]
        p1 = jnp.where(z1 >= 0, z1, 0.2 * z1)
        wb = w[0]
        hp = jnp.dot(p0, wb[:Cp2, :], preferred_element_type=jnp.float32)
        hp = hp + jnp.dot(p1, wb[Cp2:, :], preferred_element_type=jnp.float32)
        hp_ref[...] = hp[None]
        att_s = asr[0]
        att_d = adr[0]
        a_s = jnp.stack([
            jnp.sum(hp[:, :Cout] * att_s[0:1, :], axis=1),
            jnp.sum(hp[:, Cout:] * att_s[1:2, :], axis=1)], axis=0)
        a_d = jnp.stack([
            jnp.sum(hp[:, :Cout] * att_d[0:1, :], axis=1),
            jnp.sum(hp[:, Cout:] * att_d[1:2, :], axis=1)], axis=0)
        vb = mref[0] > 0.5
        as_ref[...] = jnp.where(vb, a_s, NEG)[None]
        ad_ref[...] = jnp.where(vb, a_d, NEG)[None]

    Cp = 2 * Cp2
    return pl.pallas_call(
        body,
        grid=(2, 8),
        in_specs=[
            pl.BlockSpec((ROWS, Cp2), lambda h, i: (i, 0)),
            pl.BlockSpec((ROWS, Cp2), lambda h, i: (8 + i, 0)),
            pl.BlockSpec((1, 1, ROWS), lambda h, i: (i, 0, 0)),
            pl.BlockSpec((1, Cp), lambda h, i: (0, 0)),
            pl.BlockSpec((1, Cp, C2), lambda h, i: (h, 0, 0)),
            pl.BlockSpec((1, 2, Cout), lambda h, i: (h, 0, 0)),
            pl.BlockSpec((1, 2, Cout), lambda h, i: (h, 0, 0)),
        ],
        out_specs=[
            pl.BlockSpec((1, ROWS, C2), lambda h, i: (h, i, 0)),
            pl.BlockSpec((1, 2, ROWS), lambda h, i: (h, 0, i)),
            pl.BlockSpec((1, 2, ROWS), lambda h, i: (h, 0, i)),
        ],
        out_shape=[
            jax.ShapeDtypeStruct((2, NP, C2), jnp.float32),
            jax.ShapeDtypeStruct((2, 2, NP), jnp.float32),
            jax.ShapeDtypeStruct((2, 2, NP), jnp.float32),
        ],
    )(acc2, acc2, mask, b_prev, W3, as3, ad3)


_MESH = plsc.VectorSubcoreMesh(core_axis_name="c", subcore_axis_name="s")
_B2 = 400
_SUP2 = 2000
_EB2 = E // 32


def _all_max(ref):
    def body(j, m):
        return jnp.maximum(m, ref[pl.ds(j * 16, 16)])
    m = lax.fori_loop(0, NP // 16, body,
                      jnp.full((16,), -3.0e38, jnp.float32))
    lane = lax.iota(jnp.int32, 16)
    dn = lax.GatherDimensionNumbers(offset_dims=(), collapsed_slice_dims=(0,),
                                    start_index_map=(0,))
    for kk in (8, 4, 2, 1):
        idx = lax.rem(lane + kk, 16)
        rot = lax.gather(m, idx[:, None], dn, (1,),
                         mode=lax.GatherScatterMode.PROMISE_IN_BOUNDS,
                         unique_indices=True)
        m = jnp.maximum(m, rot)
    return m


@functools.partial(
    pl.kernel,
    mesh=_MESH,
    compiler_params=pltpu.CompilerParams(needs_layout_passes=False),
    out_type=[
        jax.ShapeDtypeStruct((4 * E,), jnp.float32),
        jax.ShapeDtypeStruct((2 * 4 * NP,), jnp.float32),
    ],
    scratch_types=[
        pltpu.VMEM((NP,), jnp.float32),
        pltpu.VMEM((NP,), jnp.float32),
        pltpu.VMEM((NP,), jnp.float32),
        pltpu.VMEM((_SUP2,), jnp.int32),
        pltpu.VMEM((_SUP2,), jnp.int32),
        pltpu.VMEM((_SUP2,), jnp.float32),
        pltpu.VMEM((1280,), jnp.float32),
        pltpu.VMEM((1280,), jnp.float32),
        pltpu.VMEM_SHARED((16, NP), jnp.float32),
    ],
)
def _k2(asrc_hbm, adst_hbm, src_hbm, dst_hbm, exf_hbm, denp_hbm,
        asrc_vm, adst_vm, den_vm, srcv, dstv, exst, reda, redb, sp_den):
    c = lax.axis_index("c")
    s = lax.axis_index("s")
    h = lax.rem(s, 4)
    g = s // 4

    pltpu.sync_copy(asrc_hbm.at[pl.ds(h * NP, NP)], asrc_vm)
    pltpu.sync_copy(adst_hbm.at[pl.ds(h * NP, NP)], adst_vm)

    zv = jnp.zeros((16,), jnp.float32)

    def zbody(j, _):
        den_vm[pl.ds(j * 16, 16)] = zv
        return 0
    lax.fori_loop(0, NP // 16, zbody, 0)

    M = jnp.maximum(_all_max(asrc_vm) + _all_max(adst_vm), 0.0)

    t0 = c * (E // 2) + g * (E // 8)

    def chunk(k, _):
        base = t0 + k * _SUP2
        pltpu.sync_copy(src_hbm.at[pl.ds(base, _SUP2)], srcv)
        pltpu.sync_copy(dst_hbm.at[pl.ds(base, _SUP2)], dstv)

        def vec(j, _):
            s16 = srcv[pl.ds(j * 16, 16)]
            d16 = dstv[pl.ds(j * 16, 16)]
            ag = plsc.load_gather(asrc_vm, [s16])
            dg = plsc.load_gather(adst_vm, [d16])
            z = ag + dg
            e = jnp.where(z >= 0, z, 0.2 * z)
            ex = jnp.exp(e - M)
            exst[pl.ds(j * 16, 16)] = ex
            plsc.addupdate_scatter(den_vm, [d16], ex)
            return 0
        lax.fori_loop(0, _SUP2 // 16, vec, 0)
        pltpu.sync_copy(exst, exf_hbm.at[pl.ds(h * E + base, _SUP2)])
        return 0
    lax.fori_loop(0, (E // 8) // _SUP2, chunk, 0)

    pltpu.sync_copy(den_vm, sp_den.at[s])
    plsc.subcore_barrier()

    for r in range(2):
        bcol = g * 2560 + r * 1280
        pltpu.sync_copy(sp_den.at[h, pl.ds(bcol, 1280)], reda)
        for kpart in range(1, 4):
            pltpu.sync_copy(sp_den.at[h + 4 * kpart, pl.ds(bcol, 1280)],
                            redb)

            def addb(j, _):
                reda[pl.ds(j * 16, 16)] = (reda[pl.ds(j * 16, 16)]
                                           + redb[pl.ds(j * 16, 16)])
                return 0
            lax.fori_loop(0, 80, addb, 0)
        pltpu.sync_copy(
            reda, denp_hbm.at[pl.ds(c * 4 * NP + h * NP + bcol, 1280)])


_B4 = 80
_SUP = 2000
_EB4 = E // 16
_RT = NP // 16


def _make_k4(C2, Cout):
    nq = C2 // 16

    @functools.partial(
        pl.kernel,
        mesh=_MESH,
        compiler_params=pltpu.CompilerParams(
            needs_layout_passes=False, use_tc_tiling_on_sc=False),
        out_type=jax.ShapeDtypeStruct((2 * NP, C2), jnp.float32),
        scratch_types=[
            pltpu.VMEM((2 * NP,), jnp.float32),
            pltpu.VMEM((_SUP,), jnp.int32),
            pltpu.VMEM((_SUP,), jnp.int32),
            pltpu.VMEM((_SUP,), jnp.float32),
            pltpu.VMEM((_SUP,), jnp.float32),
            pltpu.VMEM((_B4,), jnp.int32),
            pltpu.VMEM((_B4,), jnp.int32),
            pltpu.VMEM((_B4,), jnp.float32),
            pltpu.VMEM((_B4,), jnp.float32),
            pltpu.VMEM((_B4, C2), jnp.float32),
            pltpu.VMEM_SHARED((NP, C2), jnp.float32),
            pltpu.SemaphoreType.DMA,
        ],
    )
    def k4(hp_hbm, src_hbm, dst_hbm, exf_hbm, invd_hbm, acc_hbm,
           invd_vm, srcB, dstB, exB0, exB1, gidx, dst_cur, alp0, alp1,
           rows_v, acc_sp, sem):
        c = lax.axis_index("c")
        s = lax.axis_index("s")

        pltpu.sync_copy(invd_hbm.at[pl.ds(c * 2 * NP, 2 * NP)], invd_vm)

        zv = jnp.zeros((16,), jnp.float32)

        def zrow(r, _):
            for q in range(nq):
                rows_v[r, pl.ds(q * 16, 16)] = zv
            return 0
        lax.fori_loop(0, _B4, zrow, 0)
        r0 = s * _RT

        def zchunk(k, _):
            pltpu.sync_copy(rows_v, acc_sp.at[pl.ds(r0 + k * _B4, _B4)])
            return 0
        lax.fori_loop(0, _RT // _B4, zchunk, 0)
        plsc.subcore_barrier()

        t0 = s * _EB4
        coff = c * NP
        exbs = [exB0, exB1]
        alps = [alp0, alp1]

        def sup(ks, _):
            base = t0 + ks * _SUP
            pltpu.sync_copy(src_hbm.at[pl.ds(base, _SUP)], srcB)
            pltpu.sync_copy(dst_hbm.at[pl.ds(base, _SUP)], dstB)
            for h2 in range(2):
                pltpu.sync_copy(
                    exf_hbm.at[pl.ds((2 * c + h2) * E + base, _SUP)],
                    exbs[h2])

            def sub(r, _):
                off = r * _B4

                def vec(j, _):
                    pos = off + j * 16
                    s16 = srcB[pl.ds(pos, 16)]
                    d16 = dstB[pl.ds(pos, 16)]
                    gidx[pl.ds(j * 16, 16)] = s16 + coff
                    dst_cur[pl.ds(j * 16, 16)] = d16
                    for h2 in range(2):
                        ivd = plsc.load_gather(invd_vm, [d16 + h2 * NP])
                        alps[h2][pl.ds(j * 16, 16)] = (
                            exbs[h2][pl.ds(pos, 16)] * ivd)
                    return 0
                lax.fori_loop(0, _B4 // 16, vec, 0)

                pltpu.async_copy(hp_hbm.at[gidx], rows_v, sem).wait()

                def scale(j, _):
                    a0v = alp0[pl.ds(j * 16, 16)]
                    a1v = alp1[pl.ds(j * 16, 16)]
                    for e2 in range(16):
                        ed = j * 16 + e2
                        a0 = a0v[e2]
                        a1 = a1v[e2]
                        if Cout >= 16:
                            for q in range(nq):
                                av = a0 if q < Cout // 16 else a1
                                rows_v[ed, pl.ds(q * 16, 16)] = (
                                    rows_v[ed, pl.ds(q * 16, 16)] * av)
                        else:
                            lane = lax.iota(jnp.int32, 16)
                            av = jnp.where(lane < Cout,
                                           jnp.broadcast_to(a0, (16,)),
                                           jnp.broadcast_to(a1, (16,)))
                            rows_v[ed, pl.ds(0, 16)] = (
                                rows_v[ed, pl.ds(0, 16)] * av)
                    return 0
                lax.fori_loop(0, _B4 // 16, scale, 0)

                pltpu.sync_copy(rows_v, acc_sp.at[dst_cur], add=True)
                return 0
            lax.fori_loop(0, _SUP // _B4, sub, 0)
            return 0
        lax.fori_loop(0, _EB4 // _SUP, sup, 0)

        plsc.subcore_barrier()

        def wchunk(k, _):
            pltpu.sync_copy(acc_sp.at[pl.ds(r0 + k * _B4, _B4)],
                            acc_hbm.at[pl.ds(coff + r0 + k * _B4, _B4)])
            return 0
        lax.fori_loop(0, _RT // _B4, wchunk, 0)

    return k4


def _k5(acc2, mask, b4, batch2, fcW, fcb):
    def body(in0, in1, mref, bref, batr, wref, cref,
             out_ref, sums, counts):
        i = pl.program_id(0)
        z0 = in0[...] + bref[:, :16]
        p0 = jnp.where(z0 >= 0, z0, 0.2 * z0)
        z1 = in1[...] + bref[:, 16:]
        p1 = jnp.where(z1 >= 0, z1, 0.2 * z1)
        p = jnp.concatenate([p0, p1], axis=1)
        valid = mref[0] > 0.5
        bf = jnp.where(valid, batr[0], jnp.int32(16))
        gids = lax.broadcasted_iota(jnp.int32, (16, ROWS), 0)
        onehot = (gids == bf).astype(jnp.float32)
        contrib = jnp.dot(onehot, p, preferred_element_type=jnp.float32)
        cnt = jnp.sum(onehot, axis=1)

        @pl.when(i == 0)
        def _():
            sums[...] = jnp.zeros_like(sums)
            counts[...] = jnp.zeros_like(counts)

        sums[...] += contrib
        counts[...] += jnp.broadcast_to(cnt[:, None], counts.shape)

        @pl.when(i == 7)
        def _():
            pooled = sums[...] / jnp.maximum(counts[:, :1], 1.0)
            out_ref[...] = (
                jnp.dot(pooled, wref[...],
                        preferred_element_type=jnp.float32) + cref[...])

    return pl.pallas_call(
        body,
        grid=(8,),
        in_specs=[
            pl.BlockSpec((ROWS, 16), lambda i: (i, 0)),
            pl.BlockSpec((ROWS, 16), lambda i: (8 + i, 0)),
            pl.BlockSpec((1, 1, ROWS), lambda i: (i, 0, 0)),
            pl.BlockSpec((1, 32), lambda i: (0, 0)),
            pl.BlockSpec((1, 1, ROWS), lambda i: (i, 0, 0)),
            pl.BlockSpec((32, 16), lambda i: (0, 0)),
            pl.BlockSpec((1, 16), lambda i: (0, 0)),
        ],
        out_specs=pl.BlockSpec((16, 16), lambda i: (0, 0)),
        out_shape=jax.ShapeDtypeStruct((16, 16), jnp.float32),
        scratch_shapes=[
            pltpu.VMEM((16, 32), jnp.float32),
            pltpu.VMEM((16, 32), jnp.float32),
        ],
    )(acc2, acc2, mask, b4, batch2, fcW, fcb)


_K4 = {256: _make_k4(128, 64), 128: _make_k4(64, 32),
       64: _make_k4(32, 16), 32: _make_k4(16, 8)}


def _split_w(W):
    Cin, C = W.shape
    return W.reshape(Cin, 2, C // 2).transpose(1, 0, 2)


def _layer(hp, asrc3, adst3, src1, dst1):
    C2 = hp.shape[2]
    asrc = asrc3.reshape(4 * NP)
    adst = adst3.reshape(4 * NP)
    exf, denp = _k2(asrc, adst, src1, dst1)
    dpair = denp.reshape(2, 4 * NP)
    den = dpair[0] + dpair[1]
    invd = 1.0 / (den + 1e-16)
    hp2 = hp.reshape(2 * NP, C2)
    return _K4[2 * C2](hp2, src1, dst1, exf, invd)


def kernel(x, edge_index, batch, W1, as1, ad1, b1, W2, as2, ad2, b2,
           W3, as3, ad3, b3, W4, as4, ad4, b4, fcW, fcb):
    x_pad = jnp.zeros((NP, x.shape[1]), jnp.float32).at[:N].set(x)
    batch2 = (jnp.zeros((NP,), jnp.int32).at[:N].set(batch)
              .reshape(8, 1, ROWS))
    src1 = edge_index[0]
    dst1 = edge_index[1]

    hp, asrc3, adst3, mask = _k1_first(
        x_pad, _split_w(W1), as1.reshape(2, 2, -1), ad1.reshape(2, 2, -1))
    acc = _layer(hp, asrc3, adst3, src1, dst1)
    hp, asrc3, adst3 = _k1_mid(
        acc, mask, b1[None, :], _split_w(W2),
        as2.reshape(2, 2, -1), ad2.reshape(2, 2, -1))
    acc = _layer(hp, asrc3, adst3, src1, dst1)
    hp, asrc3, adst3 = _k1_mid(
        acc, mask, b2[None, :], _split_w(W3),
        as3.reshape(2, 2, -1), ad3.reshape(2, 2, -1))
    acc = _layer(hp, asrc3, adst3, src1, dst1)
    hp, asrc3, adst3 = _k1_mid(
        acc, mask, b3[None, :], _split_w(W4),
        as4.reshape(2, 2, -1), ad4.reshape(2, 2, -1))
    acc = _layer(hp, asrc3, adst3, src1, dst1)

    return _k5(acc, mask, b4[None, :], batch2, fcW, fcb[None, :])
